# baseline (speedup 1.0000x reference)
import functools

import jax
import jax.numpy as jnp
from jax import lax
from jax.experimental import pallas as pl
from jax.experimental.pallas import tpu as pltpu
from jax.experimental.pallas import tpu_sc as plsc

N = 10000
E = 320000
C = 128
K = 27

NC = 2
NS = 16
NW = NC * NS

CHUNK = 128
A0 = 56
A1 = 104
TOTCH = NS * (A0 + A1)
E_PAD = TOTCH * CHUNK
N_ACC = N + 112
RPT = N_ACC // NS


def _mm_body(x_ref, w_ref, z_ref):
    z_ref[...] = jnp.dot(x_ref[...], w_ref[0],
                         preferred_element_type=jnp.float32)[None]


def _compute_z(x, W):
    return pl.pallas_call(
        _mm_body,
        grid=(K,),
        in_specs=[
            pl.BlockSpec((N, C), lambda k: (0, 0)),
            pl.BlockSpec((1, C, C), lambda k: (k, 0, 0)),
        ],
        out_specs=pl.BlockSpec((1, N, C), lambda k: (k, 0, 0)),
        out_shape=jax.ShapeDtypeStruct((K, N, C), jnp.float32),
    )(x, W)


def _sc_body(z_hbm, gidx_hbm, dst_hbm, zeros_hbm, out_hbm,
             gidx_v, dst_v, rows_v, acc):
    cid = lax.axis_index("c")
    sid = lax.axis_index("s")
    nch = A0 + cid * (A1 - A0)
    off = cid * (NS * A0) + sid * A0 + cid * sid * (A1 - A0)
    pltpu.sync_copy(gidx_hbm.at[pl.ds(off, A1)], gidx_v)
    pltpu.sync_copy(dst_hbm.at[pl.ds(off, A1)], dst_v)
    pltpu.sync_copy(zeros_hbm.at[pl.ds(sid * RPT, RPT)],
                    acc.at[pl.ds(sid * RPT, RPT)])
    plsc.subcore_barrier()

    def body(j, carry):
        pltpu.sync_copy(z_hbm.at[gidx_v.at[j]], rows_v)
        pltpu.sync_copy(rows_v, acc.at[dst_v.at[j]], add=True)
        return carry

    lax.fori_loop(0, nch, body, 0)
    plsc.subcore_barrier()
    pltpu.sync_copy(acc.at[pl.ds(sid * RPT, RPT)],
                    out_hbm.at[cid, pl.ds(sid * RPT, RPT)])


def _sc_scatter(zf, gidx, dst, zeros):
    mesh = plsc.VectorSubcoreMesh(core_axis_name="c", subcore_axis_name="s")
    fn = functools.partial(
        pl.kernel, _sc_body, mesh=mesh,
        out_type=jax.ShapeDtypeStruct((NC, N_ACC, C), jnp.float32),
        scratch_types=[
            pltpu.VMEM((A1, CHUNK), jnp.int32),
            pltpu.VMEM((A1, CHUNK), jnp.int32),
            pltpu.VMEM((CHUNK, C), jnp.float32),
            pltpu.VMEM_SHARED((N_ACC, C), jnp.float32),
        ],
    )()
    return fn(zf, gidx, dst, zeros)


def _bn_body(p_ref, g_ref, b_ref, o_ref):
    o = p_ref[0, :N, :] + p_ref[1, :N, :]
    mean = jnp.mean(o, axis=0, keepdims=True)
    cen = o - mean
    var = jnp.mean(cen * cen, axis=0, keepdims=True)
    y = cen * lax.rsqrt(var + 1e-5) * g_ref[0] + b_ref[0]
    o_ref[...] = jnp.where(y > 0, y, jnp.exp(jnp.minimum(y, 0.0)) - 1.0)


def _bn_elu(partials, gamma, beta):
    return pl.pallas_call(
        _bn_body,
        in_specs=[
            pl.BlockSpec((NC, N_ACC, C), lambda: (0, 0, 0)),
            pl.BlockSpec((1, C), lambda: (0, 0)),
            pl.BlockSpec((1, C), lambda: (0, 0)),
        ],
        out_specs=pl.BlockSpec((N, C), lambda: (0, 0)),
        out_shape=jax.ShapeDtypeStruct((N, C), jnp.float32),
    )(partials, gamma.reshape(1, C), beta.reshape(1, C))


def kernel(x, edge_index, kernel_id, W, gamma, beta):
    src = edge_index[0].astype(jnp.int32)
    dst = edge_index[1].astype(jnp.int32)
    kid = kernel_id.astype(jnp.int32)

    z = _compute_z(x, W)
    zf = z.reshape(K * N, C)

    pad = E_PAD - E
    gidx = jnp.concatenate([kid * N + src, jnp.zeros((pad,), jnp.int32)])
    dpad = N + jnp.arange(pad, dtype=jnp.int32) % (N_ACC - N)
    dstp = jnp.concatenate([dst, dpad])
    gidx = gidx.reshape(TOTCH, CHUNK)
    dstp = dstp.reshape(TOTCH, CHUNK)
    zeros = jnp.zeros((N_ACC, C), jnp.float32)

    partials = _sc_scatter(zf, gidx, dstp, zeros)
    return _bn_elu(partials, gamma, beta)


# speedup vs baseline: 1.1592x; 1.1592x over previous
import functools

import jax
import jax.numpy as jnp
from jax import lax
from jax.experimental import pallas as pl
from jax.experimental.pallas import tpu as pltpu
from jax.experimental.pallas import tpu_sc as plsc

N = 10000
E = 320000
C = 128
K = 27

NC = 2
NS = 16
NW = NC * NS

CHUNK = 128
A0 = 104
A1 = 56
AMX = max(A0, A1)
TOTCH = NS * (A0 + A1)
STGCH = TOTCH + AMX - min(A0, A1)
E_PAD = STGCH * CHUNK
N_ACC = N + 112
RPT = N_ACC // NS


def _mm_body(x_ref, w_ref, z_ref):
    z_ref[...] = jnp.dot(x_ref[...], w_ref[0],
                         preferred_element_type=jnp.float32)[None]


def _compute_z(x, W):
    return pl.pallas_call(
        _mm_body,
        grid=(K,),
        in_specs=[
            pl.BlockSpec((N, C), lambda k: (0, 0)),
            pl.BlockSpec((1, C, C), lambda k: (k, 0, 0)),
        ],
        out_specs=pl.BlockSpec((1, N, C), lambda k: (k, 0, 0)),
        out_shape=jax.ShapeDtypeStruct((K, N, C), jnp.float32),
    )(x, W)


def _sc_body(z_hbm, gidx_hbm, dst_hbm, zeros_hbm, out_hbm,
             gidx_v, dst_v, rows_v, acc):
    cid = lax.axis_index("c")
    sid = lax.axis_index("s")
    nch = A0 + cid * (A1 - A0)
    off = cid * (NS * A0) + sid * A0 + cid * sid * (A1 - A0)
    pltpu.sync_copy(gidx_hbm.at[pl.ds(off, AMX)], gidx_v)
    pltpu.sync_copy(dst_hbm.at[pl.ds(off, AMX)], dst_v)
    pltpu.sync_copy(zeros_hbm.at[pl.ds(sid * RPT, RPT)],
                    acc.at[pl.ds(sid * RPT, RPT)])
    plsc.subcore_barrier()

    def body(j, carry):
        pltpu.sync_copy(z_hbm.at[gidx_v.at[j]], rows_v)
        pltpu.sync_copy(rows_v, acc.at[dst_v.at[j]], add=True)
        return carry

    lax.fori_loop(0, nch, body, 0)
    plsc.subcore_barrier()
    pltpu.sync_copy(acc.at[pl.ds(sid * RPT, RPT)],
                    out_hbm.at[cid, pl.ds(sid * RPT, RPT)])


def _sc_scatter(zf, gidx, dst, zeros):
    mesh = plsc.VectorSubcoreMesh(core_axis_name="c", subcore_axis_name="s")
    fn = functools.partial(
        pl.kernel, _sc_body, mesh=mesh,
        out_type=jax.ShapeDtypeStruct((NC, N_ACC, C), jnp.float32),
        scratch_types=[
            pltpu.VMEM((AMX, CHUNK), jnp.int32),
            pltpu.VMEM((AMX, CHUNK), jnp.int32),
            pltpu.VMEM((CHUNK, C), jnp.float32),
            pltpu.VMEM_SHARED((N_ACC, C), jnp.float32),
        ],
    )()
    return fn(zf, gidx, dst, zeros)


def _bn_body(p_ref, g_ref, b_ref, o_ref):
    o = p_ref[0, :N, :] + p_ref[1, :N, :]
    mean = jnp.mean(o, axis=0, keepdims=True)
    cen = o - mean
    var = jnp.mean(cen * cen, axis=0, keepdims=True)
    y = cen * lax.rsqrt(var + 1e-5) * g_ref[0] + b_ref[0]
    o_ref[...] = jnp.where(y > 0, y, jnp.exp(jnp.minimum(y, 0.0)) - 1.0)


def _bn_elu(partials, gamma, beta):
    return pl.pallas_call(
        _bn_body,
        in_specs=[
            pl.BlockSpec((NC, N_ACC, C), lambda: (0, 0, 0)),
            pl.BlockSpec((1, C), lambda: (0, 0)),
            pl.BlockSpec((1, C), lambda: (0, 0)),
        ],
        out_specs=pl.BlockSpec((N, C), lambda: (0, 0)),
        out_shape=jax.ShapeDtypeStruct((N, C), jnp.float32),
    )(partials, gamma.reshape(1, C), beta.reshape(1, C))


def kernel(x, edge_index, kernel_id, W, gamma, beta):
    src = edge_index[0].astype(jnp.int32)
    dst = edge_index[1].astype(jnp.int32)
    kid = kernel_id.astype(jnp.int32)

    z = _compute_z(x, W)
    zf = z.reshape(K * N, C)

    pad = E_PAD - E
    gidx = jnp.concatenate([kid * N + src, jnp.zeros((pad,), jnp.int32)])
    dpad = N + jnp.arange(pad, dtype=jnp.int32) % (N_ACC - N)
    dstp = jnp.concatenate([dst, dpad])
    gidx = gidx.reshape(STGCH, CHUNK)
    dstp = dstp.reshape(STGCH, CHUNK)
    zeros = jnp.zeros((N_ACC, C), jnp.float32)

    partials = _sc_scatter(zf, gidx, dstp, zeros)
    return _bn_elu(partials, gamma, beta)


# speedup vs baseline: 1.5752x; 1.3589x over previous
import functools

import jax
import jax.numpy as jnp
from jax import lax
from jax.experimental import pallas as pl
from jax.experimental.pallas import tpu as pltpu
from jax.experimental.pallas import tpu_sc as plsc

N = 10000
E = 320000
C = 128
K = 27

NC = 2
NS = 16
NW = NC * NS

CHUNK = 128
E_PAD = ((E + NW * CHUNK - 1) // (NW * CHUNK)) * (NW * CHUNK)
EPW = E_PAD // NW
NCH = EPW // CHUNK
N_ACC = N + 112
RPT = N_ACC // NS


def _mm_body(x_ref, w_ref, z_ref):
    z_ref[...] = jnp.dot(x_ref[...], w_ref[0],
                         preferred_element_type=jnp.float32)[None]


def _compute_z(x, W):
    return pl.pallas_call(
        _mm_body,
        grid=(K,),
        in_specs=[
            pl.BlockSpec((N, C), lambda k: (0, 0)),
            pl.BlockSpec((1, C, C), lambda k: (k, 0, 0)),
        ],
        out_specs=pl.BlockSpec((1, N, C), lambda k: (k, 0, 0)),
        out_shape=jax.ShapeDtypeStruct((K, N, C), jnp.float32),
    )(x, W)


def _sc_body(z_hbm, gidx_hbm, dst_hbm, zeros_hbm, out_hbm,
             gidx_v, dst_v, rows_v, acc, sem):
    cid = lax.axis_index("c")
    sid = lax.axis_index("s")
    wid = sid * NC + cid
    pltpu.sync_copy(gidx_hbm.at[wid], gidx_v)
    pltpu.sync_copy(dst_hbm.at[wid], dst_v)
    pltpu.sync_copy(zeros_hbm.at[pl.ds(sid * RPT, RPT)],
                    acc.at[pl.ds(sid * RPT, RPT)])
    plsc.subcore_barrier()

    def body(j, carry):
        pltpu.async_copy(z_hbm.at[gidx_v.at[j]], rows_v, sem).wait()
        pltpu.sync_copy(rows_v, acc.at[dst_v.at[j]], add=True)
        return carry

    lax.fori_loop(0, NCH, body, 0)
    plsc.subcore_barrier()
    pltpu.sync_copy(acc.at[pl.ds(sid * RPT, RPT)],
                    out_hbm.at[cid, pl.ds(sid * RPT, RPT)])


def _sc_scatter(zf, gidx, dst, zeros):
    mesh = plsc.VectorSubcoreMesh(core_axis_name="c", subcore_axis_name="s")
    fn = functools.partial(
        pl.kernel, _sc_body, mesh=mesh,
        out_type=jax.ShapeDtypeStruct((NC, N_ACC, C), jnp.float32),
        scratch_types=[
            pltpu.VMEM((NCH, CHUNK), jnp.int32),
            pltpu.VMEM((NCH, CHUNK), jnp.int32),
            pltpu.VMEM((CHUNK, C), jnp.float32),
            pltpu.VMEM_SHARED((N_ACC, C), jnp.float32),
            pltpu.SemaphoreType.DMA,
        ],
    )()
    return fn(zf, gidx, dst, zeros)


def _bn_body(p_ref, g_ref, b_ref, o_ref):
    o = p_ref[0, :N, :] + p_ref[1, :N, :]
    mean = jnp.mean(o, axis=0, keepdims=True)
    cen = o - mean
    var = jnp.mean(cen * cen, axis=0, keepdims=True)
    y = cen * lax.rsqrt(var + 1e-5) * g_ref[0] + b_ref[0]
    o_ref[...] = jnp.where(y > 0, y, jnp.exp(jnp.minimum(y, 0.0)) - 1.0)


def _bn_elu(partials, gamma, beta):
    return pl.pallas_call(
        _bn_body,
        in_specs=[
            pl.BlockSpec((NC, N_ACC, C), lambda: (0, 0, 0)),
            pl.BlockSpec((1, C), lambda: (0, 0)),
            pl.BlockSpec((1, C), lambda: (0, 0)),
        ],
        out_specs=pl.BlockSpec((N, C), lambda: (0, 0)),
        out_shape=jax.ShapeDtypeStruct((N, C), jnp.float32),
    )(partials, gamma.reshape(1, C), beta.reshape(1, C))


def kernel(x, edge_index, kernel_id, W, gamma, beta):
    src = edge_index[0].astype(jnp.int32)
    dst = edge_index[1].astype(jnp.int32)
    kid = kernel_id.astype(jnp.int32)

    z = _compute_z(x, W)
    zf = z.reshape(K * N, C)

    pad = E_PAD - E
    gidx = jnp.concatenate([kid * N + src, jnp.zeros((pad,), jnp.int32)])
    dstp = jnp.concatenate([dst, jnp.full((pad,), N, jnp.int32)])
    gidx = gidx.reshape(NW, NCH, CHUNK)
    dstp = dstp.reshape(NW, NCH, CHUNK)
    zeros = jnp.zeros((N_ACC, C), jnp.float32)

    partials = _sc_scatter(zf, gidx, dstp, zeros)
    return _bn_elu(partials, gamma, beta)
